# baseline (speedup 1.0000x reference)
from functools import partial

import jax
import jax.numpy as jnp
from jax.experimental import pallas as pl
from jax.experimental.pallas import tpu as pltpu

_NC = 4
_REG = 4
_NO = _NC + 4 * _REG
_TB = 4
_VMEM_LIMIT = 48 * 1024 * 1024


def _silu(x):
    return x * pl.reciprocal(1.0 + jnp.exp(-x), approx=True)


def _conv_taps(x, w_ref, b_ref, ho, wo):
    tb, _, _, k = x.shape
    acc = None
    for du in range(2):
        for dv in range(2):
            patch = x[:, du:du + ho, dv:dv + wo, :].reshape(tb * ho * wo, k)
            d = jnp.dot(patch, w_ref[2 * du + dv],
                        preferred_element_type=jnp.float32)
            acc = d if acc is None else acc + d
    return acc + b_ref[...]


def _s2d_k(x):
    tb, h2, w2, c = x.shape
    r = x.reshape(tb, h2 // 2, 2, w2, c)
    outs = []
    for p in range(2):
        row = r[:, :, p].reshape(tb, h2 // 2, w2 // 2, 2, c)
        outs.append(row[:, :, :, 0])
        outs.append(row[:, :, :, 1])
    return jnp.concatenate(outs, axis=-1)


def _pad_s2d(x, tb, h, w, c):
    xp = jnp.pad(x.reshape(tb, h, w, c), ((0, 0), (1, 1), (1, 1), (0, 0)))
    return _silu(_s2d_k(xp)).astype(jnp.bfloat16)


def _fused_kernel(y1_ref, w1_ref, b1_ref, w2_ref, b2_ref, w3_ref, b3_ref,
                  h0w_ref, h0b_ref, h1w_ref, h1b_ref, anc_ref, str_ref,
                  y_ref, r0_ref, r1_ref):
    tb = y1_ref.shape[0]

    t = _conv_taps(y1_ref[...], w1_ref, b1_ref, 32, 32)
    s2 = _pad_s2d(t, tb, 32, 32, 16)
    p3 = _conv_taps(s2, w2_ref, b2_ref, 16, 16)
    p3b = _silu(p3).astype(jnp.bfloat16)
    s3 = _pad_s2d(p3, tb, 16, 16, 32)
    p4 = _silu(_conv_taps(s3, w3_ref, b3_ref, 8, 8)).astype(jnp.bfloat16)

    h0 = jnp.dot(p3b, h0w_ref[...], preferred_element_type=jnp.float32)
    h0 = (h0 + h0b_ref[...]).reshape(tb, 256, _NO)
    h1 = jnp.dot(p4, h1w_ref[...], preferred_element_type=jnp.float32)
    h1 = (h1 + h1b_ref[...]).reshape(tb, 64, _NO)

    c0 = jnp.swapaxes(h0, 1, 2)
    c1 = jnp.swapaxes(h1, 1, 2)
    r0_ref[...] = c0
    r1_ref[...] = c1
    cat = jnp.concatenate([c0, c1], axis=2)

    bins = jax.lax.broadcasted_iota(jnp.int32, (1, _REG, 1), 1)
    bins = bins.astype(jnp.float32)
    dists = []
    for side in range(4):
        logits = cat[:, side * _REG:(side + 1) * _REG, :]
        m = jnp.max(logits, axis=1, keepdims=True)
        e = jnp.exp(logits - m)
        den = jnp.sum(e, axis=1, keepdims=True)
        num = jnp.sum(e * bins, axis=1, keepdims=True)
        dists.append(num * pl.reciprocal(den, approx=True))
    lt_x, lt_y, rb_x, rb_y = dists

    anc_x = anc_ref[0:1, :][None]
    anc_y = anc_ref[1:2, :][None]
    strd = str_ref[...][None]
    x1 = anc_x - lt_x
    y1 = anc_y - lt_y
    x2 = anc_x + rb_x
    y2 = anc_y + rb_y
    cls = cat[:, 4 * _REG:, :]
    y_ref[...] = jnp.concatenate([
        (x1 + x2) * 0.5 * strd,
        (y1 + y2) * 0.5 * strd,
        (x2 - x1) * strd,
        (y2 - y1) * strd,
        pl.reciprocal(1.0 + jnp.exp(-cls), approx=True),
    ], axis=1)


def _s2d_xla(x):
    b, h2, w2, c = x.shape
    y = x.reshape(b, h2 // 2, 2, w2 // 2, 2, c).transpose(0, 1, 3, 2, 4, 5)
    return y.reshape(b, h2 // 2, w2 // 2, 4 * c)


def _tap_weights(w):
    cout, cin = w.shape[0], w.shape[1]
    w4 = jnp.zeros((cout, cin, 4, 4), jnp.float32).at[:, :, :3, :3].set(w)
    taps = []
    for du in range(2):
        for dv in range(2):
            blk = w4[:, :, 2 * du:2 * du + 2, 2 * dv:2 * dv + 2]
            taps.append(blk.transpose(2, 3, 1, 0).reshape(4 * cin, cout))
    return jnp.stack(taps, 0).astype(jnp.bfloat16)


def _head_weights(w_box, b_box, w_cls, b_cls):
    cin = w_box.shape[1]
    wm = jnp.concatenate([w_box.reshape(-1, cin), w_cls.reshape(-1, cin)],
                         axis=0).T.astype(jnp.bfloat16)
    bm = jnp.concatenate([b_box, b_cls], axis=0).reshape(1, -1)
    return wm, bm.astype(jnp.float32)


def _make_anchors(feat_shapes, offset=0.5):
    pts, strs = [], []
    for (h, w, s) in feat_shapes:
        sx = jnp.arange(w, dtype=jnp.float32) + offset
        sy = jnp.arange(h, dtype=jnp.float32) + offset
        syy, sxx = jnp.meshgrid(sy, sx, indexing="ij")
        pts.append(jnp.stack([sxx.reshape(-1), syy.reshape(-1)], axis=0))
        strs.append(jnp.full((1, h * w), float(s), dtype=jnp.float32))
    return jnp.concatenate(pts, axis=1), jnp.concatenate(strs, axis=1)


def kernel(image, c1_w, c1_b, c2_w, c2_b, c3_w, c3_b,
           cv2_0_w, cv2_0_b, cv3_0_w, cv3_0_b,
           cv2_1_w, cv2_1_b, cv3_1_w, cv3_1_b):
    B = image.shape[0]

    xin = image.transpose(0, 2, 3, 1)
    xp = jnp.pad(xin, ((0, 0), (1, 1), (1, 1), (0, 0)))
    y1 = _s2d_xla(xp).astype(jnp.bfloat16)

    w1 = _tap_weights(c1_w)
    b1 = c1_b.reshape(1, -1).astype(jnp.float32)
    w2 = _tap_weights(c2_w)
    b2 = c2_b.reshape(1, -1).astype(jnp.float32)
    w3 = _tap_weights(c3_w)
    b3 = c3_b.reshape(1, -1).astype(jnp.float32)
    h0w, h0b = _head_weights(cv2_0_w, cv2_0_b, cv3_0_w, cv3_0_b)
    h1w, h1b = _head_weights(cv2_1_w, cv2_1_b, cv3_1_w, cv3_1_b)
    anchors, strides = _make_anchors([(16, 16, 4), (8, 8, 8)])

    A = 320
    y, r0, r1 = pl.pallas_call(
        _fused_kernel,
        out_shape=(
            jax.ShapeDtypeStruct((B, 4 + _NC, A), jnp.float32),
            jax.ShapeDtypeStruct((B, _NO, 256), jnp.float32),
            jax.ShapeDtypeStruct((B, _NO, 64), jnp.float32),
        ),
        grid=(B // _TB,),
        in_specs=[
            pl.BlockSpec((_TB, 33, 33, 12), lambda i: (i, 0, 0, 0)),
            pl.BlockSpec((4, 12, 16), lambda i: (0, 0, 0)),
            pl.BlockSpec((1, 16), lambda i: (0, 0)),
            pl.BlockSpec((4, 64, 32), lambda i: (0, 0, 0)),
            pl.BlockSpec((1, 32), lambda i: (0, 0)),
            pl.BlockSpec((4, 128, 64), lambda i: (0, 0, 0)),
            pl.BlockSpec((1, 64), lambda i: (0, 0)),
            pl.BlockSpec((32, _NO), lambda i: (0, 0)),
            pl.BlockSpec((1, _NO), lambda i: (0, 0)),
            pl.BlockSpec((64, _NO), lambda i: (0, 0)),
            pl.BlockSpec((1, _NO), lambda i: (0, 0)),
            pl.BlockSpec((2, A), lambda i: (0, 0)),
            pl.BlockSpec((1, A), lambda i: (0, 0)),
        ],
        out_specs=(
            pl.BlockSpec((_TB, 4 + _NC, A), lambda i: (i, 0, 0)),
            pl.BlockSpec((_TB, _NO, 256), lambda i: (i, 0, 0)),
            pl.BlockSpec((_TB, _NO, 64), lambda i: (i, 0, 0)),
        ),
        compiler_params=pltpu.CompilerParams(
            dimension_semantics=("parallel",),
            vmem_limit_bytes=_VMEM_LIMIT),
    )(y1, w1, b1, w2, b2, w3, b3, h0w, h0b, h1w, h1b, anchors, strides)

    raw0 = r0.reshape(B, _NO, 16, 16)
    raw1 = r1.reshape(B, _NO, 8, 8)
    return (y.astype(image.dtype), raw0, raw1)


# speedup vs baseline: 1.0428x; 1.0428x over previous
from functools import partial

import jax
import jax.numpy as jnp
from jax.experimental import pallas as pl
from jax.experimental.pallas import tpu as pltpu

_NC = 4
_REG = 4
_NO = _NC + 4 * _REG
_TB = 16
_VMEM_LIMIT = 48 * 1024 * 1024


def _silu(x):
    return x * pl.reciprocal(1.0 + jnp.exp(-x), approx=True)


def _conv_taps(x, w_ref, b_ref, ho, wo):
    tb, _, _, k = x.shape
    acc = None
    for du in range(2):
        for dv in range(2):
            patch = x[:, du:du + ho, dv:dv + wo, :].reshape(tb * ho * wo, k)
            d = jnp.dot(patch, w_ref[2 * du + dv],
                        preferred_element_type=jnp.float32)
            acc = d if acc is None else acc + d
    return acc + b_ref[...]


def _s2d_k(x):
    tb, h2, w2, c = x.shape
    r = x.reshape(tb, h2 // 2, 2, w2, c)
    outs = []
    for p in range(2):
        row = r[:, :, p].reshape(tb, h2 // 2, w2 // 2, 2, c)
        outs.append(row[:, :, :, 0])
        outs.append(row[:, :, :, 1])
    return jnp.concatenate(outs, axis=-1)


def _pad_s2d(x, tb, h, w, c):
    xp = jnp.pad(x.reshape(tb, h, w, c), ((0, 0), (1, 1), (1, 1), (0, 0)))
    return _silu(_s2d_k(xp)).astype(jnp.bfloat16)


def _fused_kernel(y1_ref, w1_ref, b1_ref, w2_ref, b2_ref, w3_ref, b3_ref,
                  h0w_ref, h0b_ref, h1w_ref, h1b_ref, anc_ref, str_ref,
                  y_ref, r0_ref, r1_ref):
    tb = y1_ref.shape[0]

    t = _conv_taps(y1_ref[...], w1_ref, b1_ref, 32, 32)
    s2 = _pad_s2d(t, tb, 32, 32, 16)
    p3 = _conv_taps(s2, w2_ref, b2_ref, 16, 16)
    p3b = _silu(p3).astype(jnp.bfloat16)
    s3 = _pad_s2d(p3, tb, 16, 16, 32)
    p4 = _silu(_conv_taps(s3, w3_ref, b3_ref, 8, 8)).astype(jnp.bfloat16)

    h0 = jnp.dot(p3b, h0w_ref[...], preferred_element_type=jnp.float32)
    h0 = (h0 + h0b_ref[...]).reshape(tb, 256, _NO)
    h1 = jnp.dot(p4, h1w_ref[...], preferred_element_type=jnp.float32)
    h1 = (h1 + h1b_ref[...]).reshape(tb, 64, _NO)

    c0 = jnp.swapaxes(h0, 1, 2)
    c1 = jnp.swapaxes(h1, 1, 2)
    r0_ref[...] = c0
    r1_ref[...] = c1
    cat = jnp.concatenate([c0, c1], axis=2)

    bins = jax.lax.broadcasted_iota(jnp.int32, (1, _REG, 1), 1)
    bins = bins.astype(jnp.float32)
    dists = []
    for side in range(4):
        logits = cat[:, side * _REG:(side + 1) * _REG, :]
        m = jnp.max(logits, axis=1, keepdims=True)
        e = jnp.exp(logits - m)
        den = jnp.sum(e, axis=1, keepdims=True)
        num = jnp.sum(e * bins, axis=1, keepdims=True)
        dists.append(num * pl.reciprocal(den, approx=True))
    lt_x, lt_y, rb_x, rb_y = dists

    anc_x = anc_ref[0:1, :][None]
    anc_y = anc_ref[1:2, :][None]
    strd = str_ref[...][None]
    x1 = anc_x - lt_x
    y1 = anc_y - lt_y
    x2 = anc_x + rb_x
    y2 = anc_y + rb_y
    cls = cat[:, 4 * _REG:, :]
    y_ref[...] = jnp.concatenate([
        (x1 + x2) * 0.5 * strd,
        (y1 + y2) * 0.5 * strd,
        (x2 - x1) * strd,
        (y2 - y1) * strd,
        pl.reciprocal(1.0 + jnp.exp(-cls), approx=True),
    ], axis=1)


def _s2d_xla(x):
    b, h2, w2, c = x.shape
    y = x.reshape(b, h2 // 2, 2, w2 // 2, 2, c).transpose(0, 1, 3, 2, 4, 5)
    return y.reshape(b, h2 // 2, w2 // 2, 4 * c)


def _tap_weights(w):
    cout, cin = w.shape[0], w.shape[1]
    w4 = jnp.zeros((cout, cin, 4, 4), jnp.float32).at[:, :, :3, :3].set(w)
    taps = []
    for du in range(2):
        for dv in range(2):
            blk = w4[:, :, 2 * du:2 * du + 2, 2 * dv:2 * dv + 2]
            taps.append(blk.transpose(2, 3, 1, 0).reshape(4 * cin, cout))
    return jnp.stack(taps, 0).astype(jnp.bfloat16)


def _head_weights(w_box, b_box, w_cls, b_cls):
    cin = w_box.shape[1]
    wm = jnp.concatenate([w_box.reshape(-1, cin), w_cls.reshape(-1, cin)],
                         axis=0).T.astype(jnp.bfloat16)
    bm = jnp.concatenate([b_box, b_cls], axis=0).reshape(1, -1)
    return wm, bm.astype(jnp.float32)


def _make_anchors(feat_shapes, offset=0.5):
    pts, strs = [], []
    for (h, w, s) in feat_shapes:
        sx = jnp.arange(w, dtype=jnp.float32) + offset
        sy = jnp.arange(h, dtype=jnp.float32) + offset
        syy, sxx = jnp.meshgrid(sy, sx, indexing="ij")
        pts.append(jnp.stack([sxx.reshape(-1), syy.reshape(-1)], axis=0))
        strs.append(jnp.full((1, h * w), float(s), dtype=jnp.float32))
    return jnp.concatenate(pts, axis=1), jnp.concatenate(strs, axis=1)


def kernel(image, c1_w, c1_b, c2_w, c2_b, c3_w, c3_b,
           cv2_0_w, cv2_0_b, cv3_0_w, cv3_0_b,
           cv2_1_w, cv2_1_b, cv3_1_w, cv3_1_b):
    B = image.shape[0]

    xin = image.transpose(0, 2, 3, 1)
    xp = jnp.pad(xin, ((0, 0), (1, 1), (1, 1), (0, 0)))
    y1 = _s2d_xla(xp).astype(jnp.bfloat16)

    w1 = _tap_weights(c1_w)
    b1 = c1_b.reshape(1, -1).astype(jnp.float32)
    w2 = _tap_weights(c2_w)
    b2 = c2_b.reshape(1, -1).astype(jnp.float32)
    w3 = _tap_weights(c3_w)
    b3 = c3_b.reshape(1, -1).astype(jnp.float32)
    h0w, h0b = _head_weights(cv2_0_w, cv2_0_b, cv3_0_w, cv3_0_b)
    h1w, h1b = _head_weights(cv2_1_w, cv2_1_b, cv3_1_w, cv3_1_b)
    anchors, strides = _make_anchors([(16, 16, 4), (8, 8, 8)])

    A = 320
    y, r0, r1 = pl.pallas_call(
        _fused_kernel,
        out_shape=(
            jax.ShapeDtypeStruct((B, 4 + _NC, A), jnp.float32),
            jax.ShapeDtypeStruct((B, _NO, 256), jnp.float32),
            jax.ShapeDtypeStruct((B, _NO, 64), jnp.float32),
        ),
        grid=(B // _TB,),
        in_specs=[
            pl.BlockSpec((_TB, 33, 33, 12), lambda i: (i, 0, 0, 0)),
            pl.BlockSpec((4, 12, 16), lambda i: (0, 0, 0)),
            pl.BlockSpec((1, 16), lambda i: (0, 0)),
            pl.BlockSpec((4, 64, 32), lambda i: (0, 0, 0)),
            pl.BlockSpec((1, 32), lambda i: (0, 0)),
            pl.BlockSpec((4, 128, 64), lambda i: (0, 0, 0)),
            pl.BlockSpec((1, 64), lambda i: (0, 0)),
            pl.BlockSpec((32, _NO), lambda i: (0, 0)),
            pl.BlockSpec((1, _NO), lambda i: (0, 0)),
            pl.BlockSpec((64, _NO), lambda i: (0, 0)),
            pl.BlockSpec((1, _NO), lambda i: (0, 0)),
            pl.BlockSpec((2, A), lambda i: (0, 0)),
            pl.BlockSpec((1, A), lambda i: (0, 0)),
        ],
        out_specs=(
            pl.BlockSpec((_TB, 4 + _NC, A), lambda i: (i, 0, 0)),
            pl.BlockSpec((_TB, _NO, 256), lambda i: (i, 0, 0)),
            pl.BlockSpec((_TB, _NO, 64), lambda i: (i, 0, 0)),
        ),
        compiler_params=pltpu.CompilerParams(
            dimension_semantics=("parallel",),
            vmem_limit_bytes=_VMEM_LIMIT),
    )(y1, w1, b1, w2, b2, w3, b3, h0w, h0b, h1w, h1b, anchors, strides)

    raw0 = r0.reshape(B, _NO, 16, 16)
    raw1 = r1.reshape(B, _NO, 8, 8)
    return (y.astype(image.dtype), raw0, raw1)
